# Initial kernel scaffold; baseline (speedup 1.0000x reference)
import jax
import jax.numpy as jnp
from jax.experimental import pallas as pl


def kernel(x, edge_index, W1, b1, W2, b2, Wc, bc):
    raise NotImplementedError("write your pallas kernel here")


# speedup vs baseline: 26.8709x; 26.8709x over previous
import functools

import jax
import jax.numpy as jnp
from jax import lax
from jax.experimental import pallas as pl
from jax.experimental.pallas import tpu as pltpu
from jax.experimental.pallas import tpu_sc as plsc

N = 10000
E = 320000
D_IN = 128
D_HID = 128
PAD2 = 16

NC = 2
NS = 16
NW = NC * NS
EW = E // NW
CHUNK = 125
NCH = EW // CHUNK
ROWS_T = N // NS

_MESH = plsc.VectorSubcoreMesh(core_axis_name="c", subcore_axis_name="s")


def _seg_sum_sc(d, gather):
    scratch = [
        pltpu.VMEM((NCH, CHUNK), jnp.int32),
        pltpu.VMEM((CHUNK, d), jnp.float32),
        pltpu.VMEM_SHARED((N, d), jnp.float32),
        pltpu.SemaphoreType.DMA,
    ]
    if gather:
        scratch.insert(0, pltpu.VMEM((NCH, CHUNK), jnp.int32))

    def body(*refs):
        if gather:
            (table, srcs, dsts, zeros, out,
             src_v, dst_v, buf_v, acc_sh, sem) = refs
        else:
            (ones, dsts, zeros, out,
             dst_v, buf_v, acc_sh, sem) = refs
        cid = lax.axis_index("c")
        sid = lax.axis_index("s")
        r0 = sid * ROWS_T
        pltpu.sync_copy(zeros, acc_sh.at[pl.ds(r0, ROWS_T)])
        pltpu.sync_copy(dsts.at[cid, sid], dst_v)
        if gather:
            pltpu.sync_copy(srcs.at[cid, sid], src_v)
        else:
            pltpu.sync_copy(ones, buf_v)
        plsc.subcore_barrier()

        def step(j, carry):
            if gather:
                pltpu.async_copy(table.at[src_v.at[j]], buf_v, sem).wait()
            pltpu.sync_copy(buf_v, acc_sh.at[dst_v.at[j]], add=True)
            return carry

        lax.fori_loop(0, NCH, step, 0)
        plsc.subcore_barrier()
        pltpu.sync_copy(acc_sh.at[pl.ds(r0, ROWS_T)],
                        out.at[cid, pl.ds(r0, ROWS_T)])

    return pl.kernel(
        body,
        out_type=jax.ShapeDtypeStruct((NC, N, d), jnp.float32),
        mesh=_MESH,
        scratch_types=scratch,
        compiler_params=pltpu.CompilerParams(use_tc_tiling_on_sc=False),
    )


def _tc1_body(x_ref, w1_ref, degp_ref, hws_ref, dinv_ref):
    deg = degp_ref[0, :, 0] + degp_ref[1, :, 0] + 1.0
    dinv = lax.rsqrt(deg)
    hw = jnp.dot(x_ref[...], w1_ref[...], preferred_element_type=jnp.float32)
    hws_ref[...] = hw * dinv[:, None]
    dinv_ref[...] = dinv[:, None]


def _tc2_body(p_ref, hws1_ref, dinv_ref, b1_ref, w2_ref, hws2_ref):
    dinv = dinv_ref[...]
    h = jnp.tanh(dinv * (p_ref[0] + p_ref[1] + hws1_ref[...]) + b1_ref[...])
    hws2_ref[...] = dinv * jnp.dot(h, w2_ref[...],
                                   preferred_element_type=jnp.float32)


def _tc3_body(p_ref, hws2_ref, dinv_ref, b2_ref, wc_ref, bc_ref, out_ref):
    s = p_ref[0] + p_ref[1] + hws2_ref[...]
    emb = jnp.tanh(dinv_ref[...] * s[:, :2] + b2_ref[...])
    out_ref[...] = jax.nn.sigmoid(
        jnp.dot(emb, wc_ref[...], preferred_element_type=jnp.float32)
        + bc_ref[...])


def kernel(x, edge_index, W1, b1, W2, b2, Wc, bc):
    src = edge_index[0].reshape(NC, NS, NCH, CHUNK)
    dst = edge_index[1].reshape(NC, NS, NCH, CHUNK)

    zeros128 = jnp.zeros((ROWS_T, D_HID), jnp.float32)
    zeros16 = jnp.zeros((ROWS_T, PAD2), jnp.float32)
    ones16 = jnp.ones((CHUNK, PAD2), jnp.float32)
    W2p = jnp.concatenate(
        [W2, jnp.zeros((D_HID, PAD2 - W2.shape[1]), jnp.float32)], axis=1)

    degp = _seg_sum_sc(PAD2, gather=False)(ones16, dst, zeros16)

    bN = 2000
    grid = (N // bN,)
    hws1, dinv = pl.pallas_call(
        _tc1_body,
        grid=grid,
        in_specs=[
            pl.BlockSpec((bN, D_IN), lambda i: (i, 0)),
            pl.BlockSpec((D_IN, D_HID), lambda i: (0, 0)),
            pl.BlockSpec((NC, bN, PAD2), lambda i: (0, i, 0)),
        ],
        out_specs=[
            pl.BlockSpec((bN, D_HID), lambda i: (i, 0)),
            pl.BlockSpec((bN, 1), lambda i: (i, 0)),
        ],
        out_shape=[
            jax.ShapeDtypeStruct((N, D_HID), jnp.float32),
            jax.ShapeDtypeStruct((N, 1), jnp.float32),
        ],
    )(x, W1, degp)

    p1 = _seg_sum_sc(D_HID, gather=True)(hws1, src, dst, zeros128)

    hws2 = pl.pallas_call(
        _tc2_body,
        grid=grid,
        in_specs=[
            pl.BlockSpec((NC, bN, D_HID), lambda i: (0, i, 0)),
            pl.BlockSpec((bN, D_HID), lambda i: (i, 0)),
            pl.BlockSpec((bN, 1), lambda i: (i, 0)),
            pl.BlockSpec((1, D_HID), lambda i: (0, 0)),
            pl.BlockSpec((D_HID, PAD2), lambda i: (0, 0)),
        ],
        out_specs=pl.BlockSpec((bN, PAD2), lambda i: (i, 0)),
        out_shape=jax.ShapeDtypeStruct((N, PAD2), jnp.float32),
    )(p1, hws1, dinv, b1.reshape(1, D_HID), W2p)

    p2 = _seg_sum_sc(PAD2, gather=True)(hws2, src, dst, zeros16)

    out = pl.pallas_call(
        _tc3_body,
        grid=grid,
        in_specs=[
            pl.BlockSpec((NC, bN, PAD2), lambda i: (0, i, 0)),
            pl.BlockSpec((bN, PAD2), lambda i: (i, 0)),
            pl.BlockSpec((bN, 1), lambda i: (i, 0)),
            pl.BlockSpec((1, 2), lambda i: (0, 0)),
            pl.BlockSpec((2, 1), lambda i: (0, 0)),
            pl.BlockSpec((1, 1), lambda i: (0, 0)),
        ],
        out_specs=pl.BlockSpec((bN, 1), lambda i: (i, 0)),
        out_shape=jax.ShapeDtypeStruct((N, 1), jnp.float32),
    )(p2, hws2, dinv, b2.reshape(1, 2), Wc, bc.reshape(1, 1))

    return out


# speedup vs baseline: 37.0899x; 1.3803x over previous
import functools

import jax
import jax.numpy as jnp
from jax import lax
from jax.experimental import pallas as pl
from jax.experimental.pallas import tpu as pltpu
from jax.experimental.pallas import tpu_sc as plsc

N = 10000
E = 320000
D_IN = 128
D_HID = 128
PAD2 = 16
DEG_W = 8

NC = 2
NS = 16
NW = NC * NS
EW = E // NW
ROWS_T = N // NS

_MESH = plsc.VectorSubcoreMesh(core_axis_name="c", subcore_axis_name="s")


def _seg_sum_sc(d, gather, chunk):
    nch = EW // chunk
    scratch = [
        pltpu.VMEM((nch, chunk), jnp.int32),
        pltpu.VMEM((2, chunk, d), jnp.float32),
        pltpu.VMEM_SHARED((N, d), jnp.float32),
        pltpu.SemaphoreType.DMA,
        pltpu.SemaphoreType.DMA,
    ]
    if gather:
        scratch.insert(0, pltpu.VMEM((nch, chunk), jnp.int32))

    def body(*refs):
        if gather:
            (table, srcs, dsts, zeros, out,
             src_v, dst_v, buf_v, acc_sh, sem0, sem1) = refs
        else:
            (ones, dsts, zeros, out,
             dst_v, buf_v, acc_sh, sem0, sem1) = refs
        cid = lax.axis_index("c")
        sid = lax.axis_index("s")
        r0 = sid * ROWS_T
        pltpu.sync_copy(zeros, acc_sh.at[pl.ds(r0, ROWS_T)])
        pltpu.sync_copy(dsts.at[cid, sid], dst_v)
        if gather:
            pltpu.sync_copy(srcs.at[cid, sid], src_v)
            sems = (sem0, sem1)

            def gather_start(j, b):
                pltpu.async_copy(table.at[src_v.at[j]], buf_v.at[b], sems[b])

            def gather_wait(j, b):
                pltpu.make_async_copy(table.at[src_v.at[j]], buf_v.at[b],
                                      sems[b]).wait()

            plsc.subcore_barrier()
            gather_start(0, 0)

            def step(jj, carry):
                j = jj * 2
                gather_start(j + 1, 1)
                gather_wait(j, 0)
                pltpu.sync_copy(buf_v.at[0], acc_sh.at[dst_v.at[j]], add=True)

                @pl.when(j + 2 < nch)
                def _():
                    gather_start(j + 2, 0)

                gather_wait(j + 1, 1)
                pltpu.sync_copy(buf_v.at[1], acc_sh.at[dst_v.at[j + 1]],
                                add=True)
                return carry

            lax.fori_loop(0, nch // 2, step, 0)
        else:
            pltpu.sync_copy(ones, buf_v.at[0])
            plsc.subcore_barrier()

            def step(j, carry):
                pltpu.sync_copy(buf_v.at[0], acc_sh.at[dst_v.at[j]], add=True)
                return carry

            lax.fori_loop(0, nch, step, 0)
        plsc.subcore_barrier()
        pltpu.sync_copy(acc_sh.at[pl.ds(r0, ROWS_T)],
                        out.at[cid, pl.ds(r0, ROWS_T)])

    return pl.kernel(
        body,
        out_type=jax.ShapeDtypeStruct((NC, N, d), jnp.float32),
        mesh=_MESH,
        scratch_types=scratch,
        compiler_params=pltpu.CompilerParams(use_tc_tiling_on_sc=False),
    )


def _tc1_body(x_ref, w1_ref, degp_ref, hws_ref, dinv_ref):
    deg = degp_ref[0, :, 0] + degp_ref[1, :, 0] + 1.0
    dinv = lax.rsqrt(deg)
    hw = jnp.dot(x_ref[...], w1_ref[...], preferred_element_type=jnp.float32)
    hws_ref[...] = hw * dinv[:, None]
    dinv_ref[...] = dinv[:, None]


def _tc2_body(p_ref, hws1_ref, dinv_ref, b1_ref, w2_ref, hws2_ref):
    dinv = dinv_ref[...]
    h = jnp.tanh(dinv * (p_ref[0] + p_ref[1] + hws1_ref[...]) + b1_ref[...])
    hws2_ref[...] = dinv * jnp.dot(h, w2_ref[...],
                                   preferred_element_type=jnp.float32)


def _tc3_body(p_ref, hws2_ref, dinv_ref, b2_ref, wc_ref, bc_ref, out_ref):
    s = p_ref[0] + p_ref[1] + hws2_ref[...]
    emb = jnp.tanh(dinv_ref[...] * s[:, :2] + b2_ref[...])
    out_ref[...] = jax.nn.sigmoid(
        jnp.dot(emb, wc_ref[...], preferred_element_type=jnp.float32)
        + bc_ref[...])


def kernel(x, edge_index, W1, b1, W2, b2, Wc, bc):
    src_a = edge_index[0].reshape(NC, NS, EW // 125, 125)
    dst_a = edge_index[1].reshape(NC, NS, EW // 125, 125)
    src_b = edge_index[0].reshape(NC, NS, EW // 100, 100)
    dst_b = edge_index[1].reshape(NC, NS, EW // 100, 100)

    zeros128 = jnp.zeros((ROWS_T, D_HID), jnp.float32)
    zeros16 = jnp.zeros((ROWS_T, PAD2), jnp.float32)
    zerosd = jnp.zeros((ROWS_T, DEG_W), jnp.float32)
    onesd = jnp.ones((125, DEG_W), jnp.float32)
    W2p = jnp.concatenate(
        [W2, jnp.zeros((D_HID, PAD2 - W2.shape[1]), jnp.float32)], axis=1)

    degp = _seg_sum_sc(DEG_W, gather=False, chunk=125)(onesd, dst_a, zerosd)

    bN = 2000
    grid = (N // bN,)
    hws1, dinv = pl.pallas_call(
        _tc1_body,
        grid=grid,
        in_specs=[
            pl.BlockSpec((bN, D_IN), lambda i: (i, 0)),
            pl.BlockSpec((D_IN, D_HID), lambda i: (0, 0)),
            pl.BlockSpec((NC, bN, DEG_W), lambda i: (0, i, 0)),
        ],
        out_specs=[
            pl.BlockSpec((bN, D_HID), lambda i: (i, 0)),
            pl.BlockSpec((bN, 1), lambda i: (i, 0)),
        ],
        out_shape=[
            jax.ShapeDtypeStruct((N, D_HID), jnp.float32),
            jax.ShapeDtypeStruct((N, 1), jnp.float32),
        ],
    )(x, W1, degp)

    p1 = _seg_sum_sc(D_HID, gather=True, chunk=100)(hws1, src_b, dst_b, zeros128)

    hws2 = pl.pallas_call(
        _tc2_body,
        grid=grid,
        in_specs=[
            pl.BlockSpec((NC, bN, D_HID), lambda i: (0, i, 0)),
            pl.BlockSpec((bN, D_HID), lambda i: (i, 0)),
            pl.BlockSpec((bN, 1), lambda i: (i, 0)),
            pl.BlockSpec((1, D_HID), lambda i: (0, 0)),
            pl.BlockSpec((D_HID, PAD2), lambda i: (0, 0)),
        ],
        out_specs=pl.BlockSpec((bN, PAD2), lambda i: (i, 0)),
        out_shape=jax.ShapeDtypeStruct((N, PAD2), jnp.float32),
    )(p1, hws1, dinv, b1.reshape(1, D_HID), W2p)

    p2 = _seg_sum_sc(PAD2, gather=True, chunk=125)(hws2, src_a, dst_a, zeros16)

    out = pl.pallas_call(
        _tc3_body,
        grid=grid,
        in_specs=[
            pl.BlockSpec((NC, bN, PAD2), lambda i: (0, i, 0)),
            pl.BlockSpec((bN, PAD2), lambda i: (i, 0)),
            pl.BlockSpec((bN, 1), lambda i: (i, 0)),
            pl.BlockSpec((1, 2), lambda i: (0, 0)),
            pl.BlockSpec((2, 1), lambda i: (0, 0)),
            pl.BlockSpec((1, 1), lambda i: (0, 0)),
        ],
        out_specs=pl.BlockSpec((bN, 1), lambda i: (i, 0)),
        out_shape=jax.ShapeDtypeStruct((N, 1), jnp.float32),
    )(p2, hws2, dinv, b2.reshape(1, 2), Wc, bc.reshape(1, 1))

    return out


# speedup vs baseline: 41.0168x; 1.1059x over previous
import functools

import jax
import jax.numpy as jnp
from jax import lax
from jax.experimental import pallas as pl
from jax.experimental.pallas import tpu as pltpu
from jax.experimental.pallas import tpu_sc as plsc

N = 10000
E = 320000
D_IN = 128
D_HID = 128
PAD2 = 16
DEG_W = 8

NC = 2
NS = 16
NW = NC * NS
EW = E // NW
ROWS_T = N // NS

_MESH = plsc.VectorSubcoreMesh(core_axis_name="c", subcore_axis_name="s")


def _seg_sum_sc(d, gather, chunk):
    nch = EW // chunk
    scratch = [
        pltpu.VMEM((nch, chunk), jnp.int32),
        pltpu.VMEM((2, chunk, d), jnp.float32),
        pltpu.VMEM_SHARED((N, d), jnp.float32),
        pltpu.SemaphoreType.DMA,
        pltpu.SemaphoreType.DMA,
    ]
    if gather:
        scratch.insert(0, pltpu.VMEM((nch, chunk), jnp.int32))

    def body(*refs):
        if gather:
            (table, srcs, dsts, zeros, out,
             src_v, dst_v, buf_v, acc_sh, sem0, sem1) = refs
        else:
            (ones, dsts, zeros, out,
             dst_v, buf_v, acc_sh, sem0, sem1) = refs
        cid = lax.axis_index("c")
        sid = lax.axis_index("s")
        r0 = sid * ROWS_T
        pltpu.sync_copy(zeros, acc_sh.at[pl.ds(r0, ROWS_T)])
        pltpu.sync_copy(dsts.at[cid, sid], dst_v)
        if gather:
            pltpu.sync_copy(srcs.at[cid, sid], src_v)
            sems = (sem0, sem1)

            def gather_start(j, b):
                pltpu.async_copy(table.at[src_v.at[j]], buf_v.at[b], sems[b])

            def gather_wait(j, b):
                pltpu.make_async_copy(table.at[src_v.at[j]], buf_v.at[b],
                                      sems[b]).wait()

            plsc.subcore_barrier()
            gather_start(0, 0)

            def step(jj, carry):
                j = jj * 2
                gather_start(j + 1, 1)
                gather_wait(j, 0)
                pltpu.sync_copy(buf_v.at[0], acc_sh.at[dst_v.at[j]], add=True)

                @pl.when(j + 2 < nch)
                def _():
                    gather_start(j + 2, 0)

                gather_wait(j + 1, 1)
                pltpu.sync_copy(buf_v.at[1], acc_sh.at[dst_v.at[j + 1]],
                                add=True)
                return carry

            lax.fori_loop(0, nch // 2, step, 0)
        else:
            pltpu.sync_copy(ones, buf_v.at[0])
            plsc.subcore_barrier()

            def step(j, carry):
                pltpu.sync_copy(buf_v.at[0], acc_sh.at[dst_v.at[j]], add=True)
                return carry

            lax.fori_loop(0, nch, step, 0)
        plsc.subcore_barrier()
        pltpu.sync_copy(acc_sh.at[pl.ds(r0, ROWS_T)],
                        out.at[cid, pl.ds(r0, ROWS_T)])

    return pl.kernel(
        body,
        out_type=jax.ShapeDtypeStruct((NC, N, d), jnp.float32),
        mesh=_MESH,
        scratch_types=scratch,
        compiler_params=pltpu.CompilerParams(use_tc_tiling_on_sc=False),
    )


def _deg_sc_body(dsts, zn, out, dst_v, acc_v):
    cid = lax.axis_index("c")
    sid = lax.axis_index("s")
    pltpu.sync_copy(zn, acc_v)
    pltpu.sync_copy(dsts.at[cid, sid], dst_v)
    ones = jnp.ones((16,), jnp.float32)

    def step(i, carry):
        d16 = dst_v[pl.ds(i * 16, 16)]
        plsc.addupdate_scatter(acc_v, [d16], ones)
        return carry

    lax.fori_loop(0, EW // 16, step, 0)
    pltpu.sync_copy(acc_v, out.at[cid, sid])


_deg_sc = pl.kernel(
    _deg_sc_body,
    out_type=jax.ShapeDtypeStruct((NC, NS, N), jnp.float32),
    mesh=_MESH,
    scratch_types=[
        pltpu.VMEM((EW,), jnp.int32),
        pltpu.VMEM((N,), jnp.float32),
    ],
    compiler_params=pltpu.CompilerParams(use_tc_tiling_on_sc=False,
                                         needs_layout_passes=False),
)


def _agg2_sc_body(c0, c1, srcs, dsts, zn, out,
                  c0_v, c1_v, src_v, dst_v, a0_v, a1_v):
    cid = lax.axis_index("c")
    sid = lax.axis_index("s")
    pltpu.sync_copy(zn, a0_v)
    pltpu.sync_copy(zn, a1_v)
    pltpu.sync_copy(c0, c0_v)
    pltpu.sync_copy(c1, c1_v)
    pltpu.sync_copy(srcs.at[cid, sid], src_v)
    pltpu.sync_copy(dsts.at[cid, sid], dst_v)

    def step(i, carry):
        s16 = src_v[pl.ds(i * 16, 16)]
        d16 = dst_v[pl.ds(i * 16, 16)]
        plsc.addupdate_scatter(a0_v, [d16], plsc.load_gather(c0_v, [s16]))
        plsc.addupdate_scatter(a1_v, [d16], plsc.load_gather(c1_v, [s16]))
        return carry

    lax.fori_loop(0, EW // 16, step, 0)
    pltpu.sync_copy(a0_v, out.at[cid, sid, 0])
    pltpu.sync_copy(a1_v, out.at[cid, sid, 1])


_agg2_sc = pl.kernel(
    _agg2_sc_body,
    out_type=jax.ShapeDtypeStruct((NC, NS, 2, N), jnp.float32),
    mesh=_MESH,
    scratch_types=[
        pltpu.VMEM((N,), jnp.float32),
        pltpu.VMEM((N,), jnp.float32),
        pltpu.VMEM((EW,), jnp.int32),
        pltpu.VMEM((EW,), jnp.int32),
        pltpu.VMEM((N,), jnp.float32),
        pltpu.VMEM((N,), jnp.float32),
    ],
    compiler_params=pltpu.CompilerParams(use_tc_tiling_on_sc=False,
                                         needs_layout_passes=False),
)


def _tc1_body(x_ref, w1_ref, degp_ref, hws_ref, dinv_ref):
    deg = degp_ref[...].sum(axis=(0, 1)) + 1.0
    dinv = lax.rsqrt(deg)
    hw = jnp.dot(x_ref[...], w1_ref[...], preferred_element_type=jnp.float32)
    hws_ref[...] = hw * dinv[:, None]
    dinv_ref[...] = dinv[:, None]


def _tc2_body(p_ref, hws1_ref, dinv_ref, b1_ref, w2_ref, c0_ref, c1_ref):
    dinv = dinv_ref[...]
    h = jnp.tanh(dinv * (p_ref[0] + p_ref[1] + hws1_ref[...]) + b1_ref[...])
    hw2 = dinv * jnp.dot(h, w2_ref[...], preferred_element_type=jnp.float32)
    c0_ref[...] = hw2[:, 0:1]
    c1_ref[...] = hw2[:, 1:2]


def _tc3_body(p_ref, c0_ref, c1_ref, dinv_ref, b2_ref, wc_ref, bc_ref,
              out_ref):
    psum = p_ref[...].sum(axis=(0, 1))
    dinv = dinv_ref[...][:, 0]
    e0 = jnp.tanh(dinv * (psum[0] + c0_ref[...][:, 0]) + b2_ref[0, 0])
    e1 = jnp.tanh(dinv * (psum[1] + c1_ref[...][:, 0]) + b2_ref[0, 1])
    out_ref[...] = jax.nn.sigmoid(
        e0 * wc_ref[0, 0] + e1 * wc_ref[1, 0] + bc_ref[0, 0])[:, None]


def kernel(x, edge_index, W1, b1, W2, b2, Wc, bc):
    src_b = edge_index[0].reshape(NC, NS, EW // 100, 100)
    dst_b = edge_index[1].reshape(NC, NS, EW // 100, 100)
    src_f = edge_index[0].reshape(NC, NS, EW)
    dst_f = edge_index[1].reshape(NC, NS, EW)

    zeros128 = jnp.zeros((ROWS_T, D_HID), jnp.float32)
    zn = jnp.zeros((N,), jnp.float32)
    W2p = jnp.concatenate(
        [W2, jnp.zeros((D_HID, PAD2 - W2.shape[1]), jnp.float32)], axis=1)

    degp = _deg_sc(dst_f, zn)

    hws1, dinv = pl.pallas_call(
        _tc1_body,
        out_shape=[
            jax.ShapeDtypeStruct((N, D_HID), jnp.float32),
            jax.ShapeDtypeStruct((N, 1), jnp.float32),
        ],
    )(x, W1, degp)

    p1 = _seg_sum_sc(D_HID, gather=True, chunk=100)(hws1, src_b, dst_b, zeros128)

    c0, c1 = pl.pallas_call(
        _tc2_body,
        out_shape=[
            jax.ShapeDtypeStruct((N, 1), jnp.float32),
            jax.ShapeDtypeStruct((N, 1), jnp.float32),
        ],
    )(p1, hws1, dinv, b1.reshape(1, D_HID), W2p)

    p2 = _agg2_sc(c0.reshape(N), c1.reshape(N), src_f, dst_f, zn)

    out = pl.pallas_call(
        _tc3_body,
        out_shape=jax.ShapeDtypeStruct((N, 1), jnp.float32),
    )(p2, c0, c1, dinv, b2.reshape(1, 2), Wc, bc.reshape(1, 1))

    return out


# speedup vs baseline: 42.1622x; 1.0279x over previous
import functools

import jax
import jax.numpy as jnp
from jax import lax
from jax.experimental import pallas as pl
from jax.experimental.pallas import tpu as pltpu
from jax.experimental.pallas import tpu_sc as plsc

N = 10000
E = 320000
D_IN = 128
D_HID = 128
PAD2 = 16
DEG_W = 8

NC = 2
NS = 16
NW = NC * NS
EW = E // NW
ROWS_T = N // NS

_MESH = plsc.VectorSubcoreMesh(core_axis_name="c", subcore_axis_name="s")


def _seg_sum_sc(d, gather, chunk, dtype=jnp.float32):
    nch = EW // chunk
    scratch = [
        pltpu.VMEM((nch, chunk), jnp.int32),
        pltpu.VMEM((2, chunk, d), dtype),
        pltpu.VMEM_SHARED((N, d), dtype),
        pltpu.SemaphoreType.DMA,
        pltpu.SemaphoreType.DMA,
    ]
    if gather:
        scratch.insert(0, pltpu.VMEM((nch, chunk), jnp.int32))

    def body(*refs):
        if gather:
            (table, srcs, dsts, zeros, out,
             src_v, dst_v, buf_v, acc_sh, sem0, sem1) = refs
        else:
            (ones, dsts, zeros, out,
             dst_v, buf_v, acc_sh, sem0, sem1) = refs
        cid = lax.axis_index("c")
        sid = lax.axis_index("s")
        r0 = sid * ROWS_T
        pltpu.sync_copy(zeros, acc_sh.at[pl.ds(r0, ROWS_T)])
        pltpu.sync_copy(dsts.at[cid, sid], dst_v)
        if gather:
            pltpu.sync_copy(srcs.at[cid, sid], src_v)
            sems = (sem0, sem1)

            def gather_start(j, b):
                pltpu.async_copy(table.at[src_v.at[j]], buf_v.at[b], sems[b])

            def gather_wait(j, b):
                pltpu.make_async_copy(table.at[src_v.at[j]], buf_v.at[b],
                                      sems[b]).wait()

            plsc.subcore_barrier()
            gather_start(0, 0)

            def step(jj, carry):
                j = jj * 2
                gather_start(j + 1, 1)
                gather_wait(j, 0)
                pltpu.sync_copy(buf_v.at[0], acc_sh.at[dst_v.at[j]], add=True)

                @pl.when(j + 2 < nch)
                def _():
                    gather_start(j + 2, 0)

                gather_wait(j + 1, 1)
                pltpu.sync_copy(buf_v.at[1], acc_sh.at[dst_v.at[j + 1]],
                                add=True)
                return carry

            lax.fori_loop(0, nch // 2, step, 0)
        else:
            pltpu.sync_copy(ones, buf_v.at[0])
            plsc.subcore_barrier()

            def step(j, carry):
                pltpu.sync_copy(buf_v.at[0], acc_sh.at[dst_v.at[j]], add=True)
                return carry

            lax.fori_loop(0, nch, step, 0)
        plsc.subcore_barrier()
        pltpu.sync_copy(acc_sh.at[pl.ds(r0, ROWS_T)],
                        out.at[cid, pl.ds(r0, ROWS_T)])

    return pl.kernel(
        body,
        out_type=jax.ShapeDtypeStruct((NC, N, d), dtype),
        mesh=_MESH,
        scratch_types=scratch,
        compiler_params=pltpu.CompilerParams(use_tc_tiling_on_sc=False),
    )


def _deg_sc_body(dsts, zn, out, dst_v, acc_v):
    cid = lax.axis_index("c")
    sid = lax.axis_index("s")
    pltpu.sync_copy(zn, acc_v)
    pltpu.sync_copy(dsts.at[cid, sid], dst_v)
    ones = jnp.ones((16,), jnp.float32)

    def step(i, carry):
        d16 = dst_v[pl.ds(i * 16, 16)]
        plsc.addupdate_scatter(acc_v, [d16], ones)
        return carry

    lax.fori_loop(0, EW // 16, step, 0)
    pltpu.sync_copy(acc_v, out.at[cid, sid])


_deg_sc = pl.kernel(
    _deg_sc_body,
    out_type=jax.ShapeDtypeStruct((NC, NS, N), jnp.float32),
    mesh=_MESH,
    scratch_types=[
        pltpu.VMEM((EW,), jnp.int32),
        pltpu.VMEM((N,), jnp.float32),
    ],
    compiler_params=pltpu.CompilerParams(use_tc_tiling_on_sc=False,
                                         needs_layout_passes=False),
)


def _agg2_sc_body(c0, c1, srcs, dsts, zn, out,
                  c0_v, c1_v, src_v, dst_v, a0_v, a1_v):
    cid = lax.axis_index("c")
    sid = lax.axis_index("s")
    pltpu.sync_copy(zn, a0_v)
    pltpu.sync_copy(zn, a1_v)
    pltpu.sync_copy(c0, c0_v)
    pltpu.sync_copy(c1, c1_v)
    pltpu.sync_copy(srcs.at[cid, sid], src_v)
    pltpu.sync_copy(dsts.at[cid, sid], dst_v)

    def step(i, carry):
        s16 = src_v[pl.ds(i * 16, 16)]
        d16 = dst_v[pl.ds(i * 16, 16)]
        plsc.addupdate_scatter(a0_v, [d16], plsc.load_gather(c0_v, [s16]))
        plsc.addupdate_scatter(a1_v, [d16], plsc.load_gather(c1_v, [s16]))
        return carry

    lax.fori_loop(0, EW // 16, step, 0)
    pltpu.sync_copy(a0_v, out.at[cid, sid, 0])
    pltpu.sync_copy(a1_v, out.at[cid, sid, 1])


_agg2_sc = pl.kernel(
    _agg2_sc_body,
    out_type=jax.ShapeDtypeStruct((NC, NS, 2, N), jnp.float32),
    mesh=_MESH,
    scratch_types=[
        pltpu.VMEM((N,), jnp.float32),
        pltpu.VMEM((N,), jnp.float32),
        pltpu.VMEM((EW,), jnp.int32),
        pltpu.VMEM((EW,), jnp.int32),
        pltpu.VMEM((N,), jnp.float32),
        pltpu.VMEM((N,), jnp.float32),
    ],
    compiler_params=pltpu.CompilerParams(use_tc_tiling_on_sc=False,
                                         needs_layout_passes=False),
)


def _tc1_body(x_ref, w1_ref, degp_ref, hws_ref, dinv_ref):
    deg = degp_ref[...].sum(axis=(0, 1)) + 1.0
    dinv = lax.rsqrt(deg)
    hw = jnp.dot(x_ref[...], w1_ref[...], preferred_element_type=jnp.float32)
    hws_ref[...] = (hw * dinv[:, None]).astype(jnp.bfloat16)
    dinv_ref[...] = dinv[:, None]


def _tc2_body(p_ref, hws1_ref, dinv_ref, b1_ref, w2_ref, c0_ref, c1_ref):
    dinv = dinv_ref[...]
    agg = (p_ref[0].astype(jnp.float32) + p_ref[1].astype(jnp.float32)
           + hws1_ref[...].astype(jnp.float32))
    h = jnp.tanh(dinv * agg + b1_ref[...])
    hw2 = dinv * jnp.dot(h, w2_ref[...], preferred_element_type=jnp.float32)
    c0_ref[...] = hw2[:, 0:1]
    c1_ref[...] = hw2[:, 1:2]


def _tc3_body(p_ref, c0_ref, c1_ref, dinv_ref, b2_ref, wc_ref, bc_ref,
              out_ref):
    psum = p_ref[...].sum(axis=(0, 1))
    dinv = dinv_ref[...][:, 0]
    e0 = jnp.tanh(dinv * (psum[0] + c0_ref[...][:, 0]) + b2_ref[0, 0])
    e1 = jnp.tanh(dinv * (psum[1] + c1_ref[...][:, 0]) + b2_ref[0, 1])
    out_ref[...] = jax.nn.sigmoid(
        e0 * wc_ref[0, 0] + e1 * wc_ref[1, 0] + bc_ref[0, 0])[:, None]


def kernel(x, edge_index, W1, b1, W2, b2, Wc, bc):
    src_b = edge_index[0].reshape(NC, NS, EW // 125, 125)
    dst_b = edge_index[1].reshape(NC, NS, EW // 125, 125)
    src_f = edge_index[0].reshape(NC, NS, EW)
    dst_f = edge_index[1].reshape(NC, NS, EW)

    zeros128 = jnp.zeros((ROWS_T, D_HID), jnp.bfloat16)
    zn = jnp.zeros((N,), jnp.float32)
    W2p = jnp.concatenate(
        [W2, jnp.zeros((D_HID, PAD2 - W2.shape[1]), jnp.float32)], axis=1)

    degp = _deg_sc(dst_f, zn)

    hws1, dinv = pl.pallas_call(
        _tc1_body,
        out_shape=[
            jax.ShapeDtypeStruct((N, D_HID), jnp.bfloat16),
            jax.ShapeDtypeStruct((N, 1), jnp.float32),
        ],
    )(x, W1, degp)

    p1 = _seg_sum_sc(D_HID, gather=True, chunk=125,
                     dtype=jnp.bfloat16)(hws1, src_b, dst_b, zeros128)

    c0, c1 = pl.pallas_call(
        _tc2_body,
        out_shape=[
            jax.ShapeDtypeStruct((N, 1), jnp.float32),
            jax.ShapeDtypeStruct((N, 1), jnp.float32),
        ],
    )(p1, hws1, dinv, b1.reshape(1, D_HID), W2p)

    p2 = _agg2_sc(c0.reshape(N), c1.reshape(N), src_f, dst_f, zn)

    out = pl.pallas_call(
        _tc3_body,
        out_shape=jax.ShapeDtypeStruct((N, 1), jnp.float32),
    )(p2, c0, c1, dinv, b2.reshape(1, 2), Wc, bc.reshape(1, 1))

    return out
